# baseline (speedup 1.0000x reference)
import functools

import jax
import jax.numpy as jnp
from jax import lax
from jax.experimental import pallas as pl
from jax.experimental.pallas import tpu as pltpu
from jax.experimental.pallas import tpu_sc as plsc

NUM_CLASSES = 100000
C, H, W = 4, 8, 8
D = C * H * W
B = 16384
NC, NS = 2, 16
NW = NC * NS
ROUNDS = D // NW
BCH = 2048
NBCH = B // BCH
LANES = 16
ITERS = BCH // LANES


def _sc_body(labels_hbm, meansT_hbm, stdsT_hbm, noiseT_hbm, outT_hbm,
             row_v, res_v, lab_v, noi_v, sem_row, sem_lab, sem_noi):
    cid = lax.axis_index("c")
    sid = lax.axis_index("s")
    wid = sid * NC + cid

    def lab_cp(t, b):
        return pltpu.async_copy(
            labels_hbm.at[pl.ds(t * BCH, BCH)], lab_v.at[b], sem_lab[b])

    def noi_cp(f, t, b):
        return pltpu.async_copy(
            noiseT_hbm.at[f, pl.ds(t * BCH, BCH)], noi_v.at[b], sem_noi[b])

    def round_body(j, carry):
        f = j * NW + wid

        cpA = pltpu.async_copy(meansT_hbm.at[f], row_v, sem_row)
        cps = [lab_cp(0, 0)]
        cpA.wait()
        for t in range(NBCH):
            if t + 1 < NBCH:
                cps.append(lab_cp(t + 1, (t + 1) & 1))
            cps[t].wait()
            b = t & 1

            def gA(i, c2):
                idx = lab_v[b, pl.ds(i * LANES, LANES)]
                res_v[pl.ds(t * BCH + i * LANES, LANES)] = (
                    plsc.load_gather(row_v, [idx]))
                return c2

            lax.fori_loop(0, ITERS, gA, 0)

        cpB = pltpu.async_copy(stdsT_hbm.at[f], row_v, sem_row)
        lcps = [lab_cp(0, 0)]
        ncps = [noi_cp(f, 0, 0)]
        cpB.wait()
        for t in range(NBCH):
            if t + 1 < NBCH:
                lcps.append(lab_cp(t + 1, (t + 1) & 1))
                ncps.append(noi_cp(f, t + 1, (t + 1) & 1))
            lcps[t].wait()
            ncps[t].wait()
            b = t & 1

            def gB(i, c2):
                idx = lab_v[b, pl.ds(i * LANES, LANES)]
                s = plsc.load_gather(row_v, [idx])
                sl = pl.ds(t * BCH + i * LANES, LANES)
                res_v[sl] = res_v[sl] + s * noi_v[b, pl.ds(i * LANES, LANES)]
                return c2

            lax.fori_loop(0, ITERS, gB, 0)

        pltpu.sync_copy(res_v, outT_hbm.at[f])
        return carry

    lax.fori_loop(0, ROUNDS, round_body, 0)


@functools.partial(jax.jit)
def _sc_call(labels, meansT, stdsT, noiseT):
    f = functools.partial(
        pl.kernel,
        out_type=jax.ShapeDtypeStruct((D, B), jnp.float32),
        mesh=plsc.VectorSubcoreMesh(
            core_axis_name="c", subcore_axis_name="s",
            num_cores=NC, num_subcores=NS),
        compiler_params=pltpu.CompilerParams(needs_layout_passes=False),
        scratch_types=[
            pltpu.VMEM((NUM_CLASSES,), jnp.float32),
            pltpu.VMEM((B,), jnp.float32),
            pltpu.VMEM((2, BCH), jnp.int32),
            pltpu.VMEM((2, BCH), jnp.float32),
            pltpu.SemaphoreType.DMA,
            (pltpu.SemaphoreType.DMA, pltpu.SemaphoreType.DMA),
            (pltpu.SemaphoreType.DMA, pltpu.SemaphoreType.DMA),
        ],
    )(_sc_body)
    return f(labels, meansT, stdsT, noiseT)


def kernel(labels, class_means, class_stds, noise):
    meansT = class_means.transpose(1, 2, 3, 0).reshape(D, NUM_CLASSES)
    stdsT = class_stds.transpose(1, 2, 3, 0).reshape(D, NUM_CLASSES)
    noiseT = noise.transpose(1, 2, 3, 0).reshape(D, B)
    outT = _sc_call(labels.astype(jnp.int32), meansT, stdsT, noiseT)
    return outT.reshape(C, H, W, B).transpose(3, 0, 1, 2)
